# Initial kernel scaffold; baseline (speedup 1.0000x reference)
import jax
import jax.numpy as jnp
from jax.experimental import pallas as pl


def kernel(indices, table):
    raise NotImplementedError("write your pallas kernel here")


# speedup vs baseline: 1.8169x; 1.8169x over previous
import functools

import jax
import jax.numpy as jnp
from jax import lax
from jax.experimental import pallas as pl
from jax.experimental.pallas import tpu as pltpu
from jax.experimental.pallas import tpu_sc as plsc

_D = 8192


@functools.lru_cache(maxsize=None)
def _make_kernel(n_lookups, d):
    info = plsc.get_sparse_core_info()
    nw = info.num_cores * info.num_subcores
    b_per_w = n_lookups // nw
    k = 8
    n_chunks = b_per_w // k

    mesh = plsc.VectorSubcoreMesh(core_axis_name="c", subcore_axis_name="s")

    @functools.partial(
        pl.kernel,
        mesh=mesh,
        out_type=jax.ShapeDtypeStruct((n_lookups, d), jnp.float32),
        scratch_types=[
            pltpu.VMEM((n_chunks, k), jnp.int32),
            pltpu.VMEM((k, d), jnp.float32),
            pltpu.SemaphoreType.DMA,
        ],
    )
    def kern(idx_hbm, table_hbm, out_hbm, idx_v, rows_v, sem):
        wid = lax.axis_index("s") * info.num_cores + lax.axis_index("c")
        base = wid * b_per_w
        pltpu.sync_copy(idx_hbm.at[wid], idx_v)

        def body(c, carry):
            pltpu.async_copy(table_hbm.at[idx_v.at[c]], rows_v, sem).wait()
            pltpu.sync_copy(rows_v, out_hbm.at[pl.ds(base + c * k, k)])
            return carry

        lax.fori_loop(0, n_chunks, body, 0)

    return kern, nw, n_chunks, k


def kernel(indices, table):
    b, t = indices.shape
    n_lookups = b * t
    kern, nw, n_chunks, k = _make_kernel(n_lookups, table.shape[1])
    idx = indices.reshape(nw, n_chunks, k).astype(jnp.int32)
    out = kern(idx, table)
    return out.reshape(b, t, table.shape[1])


# speedup vs baseline: 1.9375x; 1.0664x over previous
import functools

import jax
import jax.numpy as jnp
from jax import lax
from jax.experimental import pallas as pl
from jax.experimental.pallas import tpu as pltpu
from jax.experimental.pallas import tpu_sc as plsc

_K = 4
_IPAD = 8


@functools.lru_cache(maxsize=None)
def _make_kernel(n_lookups, d):
    info = plsc.get_sparse_core_info()
    nw = info.num_cores * info.num_subcores
    b_per_w = n_lookups // nw
    n_chunks = b_per_w // _K

    mesh = plsc.VectorSubcoreMesh(core_axis_name="c", subcore_axis_name="s")

    @functools.partial(
        pl.kernel,
        mesh=mesh,
        out_type=jax.ShapeDtypeStruct((n_lookups, d), jnp.float32),
        scratch_types=[
            pltpu.VMEM((n_chunks, _IPAD), jnp.int32),
            pltpu.VMEM((_K, d), jnp.float32),
            pltpu.VMEM((_K, d), jnp.float32),
            pltpu.SemaphoreType.DMA,
            pltpu.SemaphoreType.DMA,
        ],
    )
    def kern(idx_hbm, table_hbm, out_hbm, idx_v, rows_a, rows_b, gsem, ssem):
        wid = lax.axis_index("s") * info.num_cores + lax.axis_index("c")
        base = wid * b_per_w
        pltpu.sync_copy(idx_hbm.at[wid], idx_v)

        def start_gather(c, buf):
            pltpu.async_copy(
                table_hbm.at[idx_v.at[c, pl.ds(0, _K)]], buf, gsem
            )

        def gwait(buf):
            pltpu.make_async_copy(table_hbm.at[pl.ds(0, _K)], buf, gsem).wait()

        def scatter(c, buf):
            pltpu.async_copy(buf, out_hbm.at[pl.ds(base + c * _K, _K)], ssem)

        def swait(c, buf):
            pltpu.make_async_copy(
                buf, out_hbm.at[pl.ds(base + c * _K, _K)], ssem
            ).wait()

        start_gather(0, rows_a)

        def body(i, carry):
            c0 = 2 * i
            gwait(rows_a)
            start_gather(c0 + 1, rows_b)
            scatter(c0, rows_a)
            gwait(rows_b)
            swait(c0, rows_a)
            start_gather(c0 + 2, rows_a)
            scatter(c0 + 1, rows_b)
            swait(c0 + 1, rows_b)
            return carry

        lax.fori_loop(0, n_chunks // 2 - 1, body, 0)

        c0 = n_chunks - 2
        gwait(rows_a)
        start_gather(c0 + 1, rows_b)
        scatter(c0, rows_a)
        gwait(rows_b)
        swait(c0, rows_a)
        scatter(c0 + 1, rows_b)
        swait(c0 + 1, rows_b)

    return kern, nw, n_chunks


def kernel(indices, table):
    b, t = indices.shape
    n_lookups = b * t
    kern, nw, n_chunks = _make_kernel(n_lookups, table.shape[1])
    idx = indices.reshape(nw, n_chunks, _K).astype(jnp.int32)
    idx = jnp.pad(idx, ((0, 0), (0, 0), (0, _IPAD - _K)))
    out = kern(idx, table)
    return out.reshape(b, t, table.shape[1])
